# baseline (speedup 1.0000x reference)
import jax
import jax.numpy as jnp
from jax import lax
from jax.experimental import pallas as pl
from jax.experimental.pallas import tpu as pltpu
from jax.experimental.pallas import tpu_sc as plsc

N = 10000
NP = 10240
D = 128
DC = 64
E = 320000
NC, NS = 2, 16
NW = NC * NS
EPW = E // NW
K = 80
NB = EPW // K
RPT = NP // NS
CPT = NP // NS
NBUF = 3
NDX = 6
NQ = NB // NDX
NREM = NB - NQ * NDX

_mesh = plsc.VectorSubcoreMesh(core_axis_name="c", subcore_axis_name="s")


def _make_sc_layer(compute_cnt: bool):
    out_type = [jax.ShapeDtypeStruct((NC, NP, DC), jnp.float32)]
    if compute_cnt:
        out_type.append(jax.ShapeDtypeStruct((NC * NP,), jnp.float32))

    scratch = [
        pltpu.VMEM_SHARED((NP, DC), jnp.float32),
        pltpu.VMEM((EPW,), jnp.int32),
        pltpu.VMEM((NDX, K), jnp.int32),
        pltpu.VMEM((NBUF, K, DC), jnp.float32),
        pltpu.SemaphoreType.DMA,
    ] + [pltpu.SemaphoreType.DMA] * (NBUF + NBUF + NDX)
    if compute_cnt:
        scratch += [
            pltpu.VMEM_SHARED((NP,), jnp.float32),
            pltpu.VMEM((K,), jnp.float32),
            pltpu.VMEM((CPT,), jnp.float32),
            pltpu.SemaphoreType.DMA,
        ]

    def body(h_hbm, src_hbm, dst_hbm, *rest):
        if compute_cnt:
            (p_hbm, c_hbm, acc, sidx, didx, rows, sem_i, *sems) = rest
            *sems, cnt, ones, cbuf, sem_c = sems
        else:
            p_hbm, acc, sidx, didx, rows, sem_i, *sems = rest
        sem_g = sems[:NBUF]
        sem_s = sems[NBUF:2 * NBUF]
        sem_d = sems[2 * NBUF:]
        cid = lax.axis_index("c")
        sid = lax.axis_index("s")
        wid = sid * NC + cid

        r0 = sid * RPT
        init_cp = pltpu.async_copy(h_hbm.at[pl.ds(r0, RPT)],
                                   acc.at[pl.ds(r0, RPT)], sem_i)

        pltpu.sync_copy(src_hbm.at[wid], sidx)
        for g0 in range(4):
            pltpu.async_copy(dst_hbm.at[wid, g0], didx.at[pl.ds(g0, 1)],
                             sem_d[g0])

        for b in range(2):
            pltpu.async_copy(h_hbm.at[sidx.at[pl.ds(b * K, K)]], rows.at[b],
                             sem_g[b])

        if compute_cnt:
            for j in range(CPT // 16):
                cbuf[pl.ds(j * 16, 16)] = jnp.zeros((16,), jnp.float32)
            pltpu.sync_copy(cbuf, cnt.at[pl.ds(sid * CPT, CPT)])
            for j in range(K // 16):
                ones[pl.ds(j * 16, 16)] = jnp.ones((16,), jnp.float32)

        init_cp.wait()
        plsc.subcore_barrier()

        def issue_gather(g, b):
            pltpu.async_copy(h_hbm.at[sidx.at[pl.ds(g * K, K)]], rows.at[b],
                             sem_g[b])

        def step(g, j):
            b = j % NBUF
            d = j % NDX
            d4 = (j + 4) % NDX
            b1 = (j + NBUF - 1) % NBUF
            b2 = (j + 2) % NBUF
            pltpu.make_async_copy(h_hbm.at[sidx.at[pl.ds(g * K, K)]],
                                  rows.at[b], sem_g[b]).wait()
            pltpu.make_async_copy(dst_hbm.at[wid, g], didx.at[pl.ds(d, 1)],
                                  sem_d[d]).wait()
            pltpu.async_copy(rows.at[b], acc.at[didx.at[d]], sem_s[b],
                             add=True)
            if compute_cnt:
                @pl.when(g >= 2)
                def _():
                    pltpu.make_async_copy(ones, cnt.at[didx.at[d]],
                                          sem_c).wait()
                pltpu.async_copy(ones, cnt.at[didx.at[d]], sem_c, add=True)

            @pl.when(g + 4 < NB)
            def _():
                pltpu.async_copy(dst_hbm.at[wid, g + 4],
                                 didx.at[pl.ds(d4, 1)], sem_d[d4])

            @pl.when(g >= 1)
            def _():
                pltpu.make_async_copy(rows.at[b1], acc.at[didx.at[d]],
                                      sem_s[b1]).wait()

            @pl.when(g + 2 < NB)
            def _():
                issue_gather(g + 2, b2)

        def round_(q, _):
            for j in range(NDX):
                step(q * NDX + j, j)
            return 0

        lax.fori_loop(0, NQ, round_, 0)
        for r in range(NREM):
            step(NQ * NDX + r, r)

        pltpu.make_async_copy(rows.at[(NB - 1) % NBUF],
                              acc.at[didx.at[0]],
                              sem_s[(NB - 1) % NBUF]).wait()
        if compute_cnt:
            for _ in range(2):
                pltpu.make_async_copy(ones, cnt.at[didx.at[0]], sem_c).wait()

        plsc.subcore_barrier()

        pltpu.sync_copy(acc.at[pl.ds(r0, RPT)], p_hbm.at[cid, pl.ds(r0, RPT)])

        if compute_cnt:
            pltpu.sync_copy(cnt.at[pl.ds(sid * CPT, CPT)], cbuf)
            pltpu.sync_copy(cbuf, c_hbm.at[pl.ds(cid * NP + sid * CPT, CPT)])

    return pl.kernel(body, out_type=tuple(out_type), mesh=_mesh,
                     scratch_types=scratch,
                     compiler_params=pltpu.CompilerParams(
                         use_tc_tiling_on_sc=False))


_sc_layer_cnt = _make_sc_layer(True)
_sc_layer = _make_sc_layer(False)


_MMBLK = 1280


def _mm_body(x_ref, w_ref, b_ref, oa_ref, ob_ref):
    h = lax.dot_general(
        x_ref[...], w_ref[...], (((1,), (1,)), ((), ())),
        preferred_element_type=jnp.float32) + b_ref[...]
    oa_ref[...] = h[:, :DC]
    ob_ref[...] = h[:, DC:]


def _matmul(x, w, b2):
    return pl.pallas_call(
        _mm_body,
        grid=(NP // _MMBLK,),
        in_specs=[
            pl.BlockSpec((_MMBLK, D), lambda i: (i, 0)),
            pl.BlockSpec((D, D), lambda i: (0, 0)),
            pl.BlockSpec((1, D), lambda i: (0, 0)),
        ],
        out_specs=[pl.BlockSpec((_MMBLK, DC), lambda i: (i, 0)),
                   pl.BlockSpec((_MMBLK, DC), lambda i: (i, 0))],
        out_shape=[jax.ShapeDtypeStruct((NP, DC), jnp.float32),
                   jax.ShapeDtypeStruct((NP, DC), jnp.float32)],
    )(x, w, b2)


def _comb_body(p_ref, h_ref, c_ref, o_ref):
    i = pl.program_id(0)
    c0 = c_ref[pl.ds(i * _MMBLK, _MMBLK)]
    c1 = c_ref[pl.ds(NP + i * _MMBLK, _MMBLK)]
    cnt = jnp.maximum(c0 + c1 + 1.0, 1.0)
    acc = p_ref[0] + p_ref[1] - h_ref[...]
    o_ref[...] = acc / cnt[:, None]


def _combine(p, h, c):
    return pl.pallas_call(
        _comb_body,
        grid=(NP // _MMBLK,),
        in_specs=[
            pl.BlockSpec((NC, _MMBLK, DC), lambda i: (0, i, 0)),
            pl.BlockSpec((_MMBLK, DC), lambda i: (i, 0)),
            pl.BlockSpec((NC * NP,), lambda i: (0,)),
        ],
        out_specs=pl.BlockSpec((_MMBLK, DC), lambda i: (i, 0)),
        out_shape=jax.ShapeDtypeStruct((NP, DC), jnp.float32),
    )(p, h, c)


def kernel(x, edge_index, W, b):
    dst = edge_index[0].reshape(NW, NB, 1, K)
    src = edge_index[1].reshape(NW, EPW)
    xp = jnp.pad(x, ((0, NP - N), (0, 0)))
    h0a, h0b = _matmul(xp, W, b.reshape(1, D))
    p1a, c = _sc_layer_cnt(h0a, src, dst)
    (p1b,) = _sc_layer(h0b, src, dst)
    h1a = _combine(p1a, h0a, c)
    h1b = _combine(p1b, h0b, c)
    (p2a,) = _sc_layer(h1a, src, dst)
    (p2b,) = _sc_layer(h1b, src, dst)
    h2a = _combine(p2a, h1a, c)
    h2b = _combine(p2b, h1b, c)
    return jnp.concatenate([h2a[:N], h2b[:N]], axis=1)


# speedup vs baseline: 1.3341x; 1.3341x over previous
import jax
import jax.numpy as jnp
from jax import lax
from jax.experimental import pallas as pl
from jax.experimental.pallas import tpu as pltpu
from jax.experimental.pallas import tpu_sc as plsc

N = 10000
NP = 10240
D = 128
E = 320000
NC, NS = 2, 16
NW = NC * NS
EPW = E // NW
K = 80
NB = EPW // K
RPT = NP // NS
CH = 80
NCP = RPT // CH
CPT = NP // NS
NBUF = 3
NDX = 6
NQ = NB // NDX
NREM = NB - NQ * NDX

_mesh = plsc.VectorSubcoreMesh(core_axis_name="c", subcore_axis_name="s")


def _make_sc_layer(compute_cnt: bool):
    out_type = [jax.ShapeDtypeStruct((NC, NP, D), jnp.float32)]
    if compute_cnt:
        out_type.append(jax.ShapeDtypeStruct((NC * NP,), jnp.float32))

    scratch = [
        pltpu.VMEM_SHARED((NP, D), jnp.float32),
        pltpu.VMEM((EPW,), jnp.int32),
        pltpu.VMEM((NDX, K), jnp.int32),
        pltpu.VMEM((NBUF, K, D), jnp.float32),
        pltpu.SemaphoreType.DMA,
    ] + [pltpu.SemaphoreType.DMA] * (NBUF + NBUF + NDX)
    if compute_cnt:
        scratch += [
            pltpu.VMEM_SHARED((NP,), jnp.float32),
            pltpu.VMEM((K,), jnp.float32),
            pltpu.VMEM((CPT,), jnp.float32),
            pltpu.SemaphoreType.DMA,
        ]

    def body(h_hbm, src_hbm, dst_hbm, *rest):
        if compute_cnt:
            (p_hbm, c_hbm, acc, sidx, didx, rows, sem_i, *sems) = rest
            *sems, cnt, ones, cbuf, sem_c = sems
        else:
            p_hbm, acc, sidx, didx, rows, sem_i, *sems = rest
        sem_g = sems[:NBUF]
        sem_s = sems[NBUF:2 * NBUF]
        sem_d = sems[2 * NBUF:]
        cid = lax.axis_index("c")
        sid = lax.axis_index("s")
        wid = sid * NC + cid

        r0 = sid * RPT
        init_cp = pltpu.async_copy(h_hbm.at[pl.ds(r0, RPT)],
                                   acc.at[pl.ds(r0, RPT)], sem_i)

        pltpu.sync_copy(src_hbm.at[wid], sidx)
        for g0 in range(4):
            pltpu.async_copy(dst_hbm.at[wid, g0], didx.at[pl.ds(g0, 1)],
                             sem_d[g0])

        for b in range(2):
            pltpu.async_copy(h_hbm.at[sidx.at[pl.ds(b * K, K)]], rows.at[b],
                             sem_g[b])

        if compute_cnt:
            for j in range(CPT // 16):
                cbuf[pl.ds(j * 16, 16)] = jnp.zeros((16,), jnp.float32)
            pltpu.sync_copy(cbuf, cnt.at[pl.ds(sid * CPT, CPT)])
            for j in range(K // 16):
                ones[pl.ds(j * 16, 16)] = jnp.ones((16,), jnp.float32)

        init_cp.wait()
        plsc.subcore_barrier()

        def issue_gather(g, b):
            pltpu.async_copy(h_hbm.at[sidx.at[pl.ds(g * K, K)]], rows.at[b],
                             sem_g[b])

        def step(g, j):
            b = j % NBUF
            d = j % NDX
            d4 = (j + 4) % NDX
            b1 = (j + NBUF - 1) % NBUF
            b2 = (j + 2) % NBUF
            pltpu.make_async_copy(h_hbm.at[sidx.at[pl.ds(g * K, K)]],
                                  rows.at[b], sem_g[b]).wait()
            pltpu.make_async_copy(dst_hbm.at[wid, g], didx.at[pl.ds(d, 1)],
                                  sem_d[d]).wait()
            pltpu.async_copy(rows.at[b], acc.at[didx.at[d]], sem_s[b],
                             add=True)
            if compute_cnt:
                @pl.when(g >= 2)
                def _():
                    pltpu.make_async_copy(ones, cnt.at[didx.at[d]],
                                          sem_c).wait()
                pltpu.async_copy(ones, cnt.at[didx.at[d]], sem_c, add=True)

            @pl.when(g + 4 < NB)
            def _():
                pltpu.async_copy(dst_hbm.at[wid, g + 4],
                                 didx.at[pl.ds(d4, 1)], sem_d[d4])

            @pl.when(g >= 1)
            def _():
                pltpu.make_async_copy(rows.at[b1], acc.at[didx.at[d]],
                                      sem_s[b1]).wait()

            @pl.when(g + 2 < NB)
            def _():
                issue_gather(g + 2, b2)

        def round_(q, _):
            for j in range(NDX):
                step(q * NDX + j, j)
            return 0

        lax.fori_loop(0, NQ, round_, 0)
        for r in range(NREM):
            step(NQ * NDX + r, r)

        pltpu.make_async_copy(rows.at[(NB - 1) % NBUF],
                              acc.at[didx.at[0]],
                              sem_s[(NB - 1) % NBUF]).wait()
        if compute_cnt:
            for _ in range(2):
                pltpu.make_async_copy(ones, cnt.at[didx.at[0]], sem_c).wait()

        plsc.subcore_barrier()

        pltpu.sync_copy(acc.at[pl.ds(r0, RPT)], p_hbm.at[cid, pl.ds(r0, RPT)])

        if compute_cnt:
            pltpu.sync_copy(cnt.at[pl.ds(sid * CPT, CPT)], cbuf)
            pltpu.sync_copy(cbuf, c_hbm.at[pl.ds(cid * NP + sid * CPT, CPT)])

    return pl.kernel(body, out_type=tuple(out_type), mesh=_mesh,
                     scratch_types=scratch)


_sc_layer_cnt = _make_sc_layer(True)
_sc_layer = _make_sc_layer(False)


_MMBLK = 1280


def _mm_body(x_ref, w_ref, b_ref, o_ref):
    o_ref[...] = lax.dot_general(
        x_ref[...], w_ref[...], (((1,), (1,)), ((), ())),
        preferred_element_type=jnp.float32) + b_ref[...]


def _matmul(x, w, b2):
    return pl.pallas_call(
        _mm_body,
        grid=(NP // _MMBLK,),
        in_specs=[
            pl.BlockSpec((_MMBLK, D), lambda i: (i, 0)),
            pl.BlockSpec((D, D), lambda i: (0, 0)),
            pl.BlockSpec((1, D), lambda i: (0, 0)),
        ],
        out_specs=pl.BlockSpec((_MMBLK, D), lambda i: (i, 0)),
        out_shape=jax.ShapeDtypeStruct((NP, D), jnp.float32),
    )(x, w, b2)


def _comb_body(p_ref, h_ref, c_ref, o_ref):
    i = pl.program_id(0)
    c0 = c_ref[pl.ds(i * _MMBLK, _MMBLK)]
    c1 = c_ref[pl.ds(NP + i * _MMBLK, _MMBLK)]
    cnt = jnp.maximum(c0 + c1 + 1.0, 1.0)
    acc = p_ref[0] + p_ref[1] - h_ref[...]
    o_ref[...] = acc / cnt[:, None]


def _combine(p, h, c):
    return pl.pallas_call(
        _comb_body,
        grid=(NP // _MMBLK,),
        in_specs=[
            pl.BlockSpec((NC, _MMBLK, D), lambda i: (0, i, 0)),
            pl.BlockSpec((_MMBLK, D), lambda i: (i, 0)),
            pl.BlockSpec((NC * NP,), lambda i: (0,)),
        ],
        out_specs=pl.BlockSpec((_MMBLK, D), lambda i: (i, 0)),
        out_shape=jax.ShapeDtypeStruct((NP, D), jnp.float32),
    )(p, h, c)


def kernel(x, edge_index, W, b):
    dst = edge_index[0].reshape(NW, NB, 1, K)
    src = edge_index[1].reshape(NW, EPW)
    xp = jnp.pad(x, ((0, NP - N), (0, 0)))
    h0 = _matmul(xp, W, b.reshape(1, D))
    p1, c = _sc_layer_cnt(h0, src, dst)
    h1 = _combine(p1, h0, c)
    (p2,) = _sc_layer(h1, src, dst)
    h2 = _combine(p2, h1, c)
    return h2[:N]


# speedup vs baseline: 1.3505x; 1.0123x over previous
import jax
import jax.numpy as jnp
from jax import lax
from jax.experimental import pallas as pl
from jax.experimental.pallas import tpu as pltpu
from jax.experimental.pallas import tpu_sc as plsc

N = 10000
NP = 10240
D = 128
E = 320000
NC, NS = 2, 16
NW = NC * NS
EPW = E // NW
K = 80
NB = EPW // K
RPT = NP // NS
CH = 80
NCP = RPT // CH
CPT = NP // NS
NBUF = 3
NDX = 6
NQ = NB // NDX
NREM = NB - NQ * NDX

_mesh = plsc.VectorSubcoreMesh(core_axis_name="c", subcore_axis_name="s")


def _make_sc_layer(compute_cnt: bool):
    out_type = [jax.ShapeDtypeStruct((NC, NP, D), jnp.float32)]
    if compute_cnt:
        out_type.append(jax.ShapeDtypeStruct((NC * NP,), jnp.float32))

    scratch = [
        pltpu.VMEM_SHARED((NP, D), jnp.float32),
        pltpu.VMEM((EPW,), jnp.int32),
        pltpu.VMEM((NDX, K), jnp.int32),
        pltpu.VMEM((NBUF, K, D), jnp.float32),
        pltpu.SemaphoreType.DMA,
    ] + [pltpu.SemaphoreType.DMA] * (NBUF + NBUF + NDX)
    if compute_cnt:
        scratch += [
            pltpu.VMEM_SHARED((NP,), jnp.float32),
            pltpu.VMEM((K,), jnp.float32),
            pltpu.VMEM((CPT,), jnp.float32),
            pltpu.SemaphoreType.DMA,
        ]

    def body(h_hbm, src_hbm, dst_hbm, *rest):
        if compute_cnt:
            (p_hbm, c_hbm, acc, sidx, didx, rows, sem_i, *sems) = rest
            *sems, cnt, ones, cbuf, sem_c = sems
        else:
            p_hbm, acc, sidx, didx, rows, sem_i, *sems = rest
        sem_g = sems[:NBUF]
        sem_s = sems[NBUF:2 * NBUF]
        sem_d = sems[2 * NBUF:]
        cid = lax.axis_index("c")
        sid = lax.axis_index("s")
        wid = sid * NC + cid

        r0 = sid * RPT
        init_cp = pltpu.async_copy(h_hbm.at[pl.ds(r0, RPT)],
                                   acc.at[pl.ds(r0, RPT)], sem_i)

        pltpu.sync_copy(src_hbm.at[wid], sidx)
        for g0 in range(4):
            pltpu.async_copy(dst_hbm.at[wid, g0], didx.at[pl.ds(g0, 1)],
                             sem_d[g0])

        for b in range(2):
            pltpu.async_copy(h_hbm.at[sidx.at[pl.ds(b * K, K)]], rows.at[b],
                             sem_g[b])

        if compute_cnt:
            for j in range(CPT // 16):
                cbuf[pl.ds(j * 16, 16)] = jnp.zeros((16,), jnp.float32)
            pltpu.sync_copy(cbuf, cnt.at[pl.ds(sid * CPT, CPT)])
            for j in range(K // 16):
                ones[pl.ds(j * 16, 16)] = jnp.ones((16,), jnp.float32)

        init_cp.wait()
        plsc.subcore_barrier()

        def issue_gather(g, b):
            pltpu.async_copy(h_hbm.at[sidx.at[pl.ds(g * K, K)]], rows.at[b],
                             sem_g[b])

        def step(g, j):
            b = j % NBUF
            d = j % NDX
            d4 = (j + 4) % NDX
            b1 = (j + NBUF - 1) % NBUF
            b2 = (j + 2) % NBUF
            pltpu.make_async_copy(h_hbm.at[sidx.at[pl.ds(g * K, K)]],
                                  rows.at[b], sem_g[b]).wait()
            pltpu.make_async_copy(dst_hbm.at[wid, g], didx.at[pl.ds(d, 1)],
                                  sem_d[d]).wait()
            pltpu.async_copy(rows.at[b], acc.at[didx.at[d]], sem_s[b],
                             add=True)
            if compute_cnt:
                @pl.when(g >= 2)
                def _():
                    pltpu.make_async_copy(ones, cnt.at[didx.at[d]],
                                          sem_c).wait()
                pltpu.async_copy(ones, cnt.at[didx.at[d]], sem_c, add=True)

            @pl.when(g + 4 < NB)
            def _():
                pltpu.async_copy(dst_hbm.at[wid, g + 4],
                                 didx.at[pl.ds(d4, 1)], sem_d[d4])

            @pl.when(g >= 1)
            def _():
                pltpu.make_async_copy(rows.at[b1], acc.at[didx.at[d]],
                                      sem_s[b1]).wait()

            @pl.when(g + 2 < NB)
            def _():
                issue_gather(g + 2, b2)

        def round_(q, _):
            for j in range(NDX):
                step(q * NDX + j, j)
            return 0

        lax.fori_loop(0, NQ, round_, 0)
        for r in range(NREM):
            step(NQ * NDX + r, r)

        pltpu.make_async_copy(rows.at[(NB - 1) % NBUF],
                              acc.at[didx.at[0]],
                              sem_s[(NB - 1) % NBUF]).wait()
        if compute_cnt:
            for _ in range(2):
                pltpu.make_async_copy(ones, cnt.at[didx.at[0]], sem_c).wait()

        plsc.subcore_barrier()

        pltpu.sync_copy(acc.at[pl.ds(r0, RPT)], p_hbm.at[cid, pl.ds(r0, RPT)])

        if compute_cnt:
            pltpu.sync_copy(cnt.at[pl.ds(sid * CPT, CPT)], cbuf)
            pltpu.sync_copy(cbuf, c_hbm.at[pl.ds(cid * NP + sid * CPT, CPT)])

    return pl.kernel(body, out_type=tuple(out_type), mesh=_mesh,
                     scratch_types=scratch)


_sc_layer_cnt = _make_sc_layer(True)
_sc_layer = _make_sc_layer(False)


_MMBLK = 1280


def _mm_body(x_ref, w_ref, b_ref, o_ref):
    o_ref[...] = lax.dot_general(
        x_ref[...], w_ref[...], (((1,), (1,)), ((), ())),
        preferred_element_type=jnp.float32) + b_ref[...]


_MMB2 = 1000


def _matmul(x, w, b2):
    return pl.pallas_call(
        _mm_body,
        grid=(N // _MMB2,),
        in_specs=[
            pl.BlockSpec((_MMB2, D), lambda i: (i, 0)),
            pl.BlockSpec((D, D), lambda i: (0, 0)),
            pl.BlockSpec((1, D), lambda i: (0, 0)),
        ],
        out_specs=pl.BlockSpec((_MMB2, D), lambda i: (i, 0)),
        out_shape=jax.ShapeDtypeStruct((NP, D), jnp.float32),
    )(x, w, b2)


def _comb_body(p_ref, h_ref, c_ref, o_ref):
    i = pl.program_id(0)
    c0 = c_ref[pl.ds(i * _MMBLK, _MMBLK)]
    c1 = c_ref[pl.ds(NP + i * _MMBLK, _MMBLK)]
    cnt = jnp.maximum(c0 + c1 + 1.0, 1.0)
    acc = p_ref[0] + p_ref[1] - h_ref[...]
    o_ref[...] = acc / cnt[:, None]


def _combine(p, h, c):
    return pl.pallas_call(
        _comb_body,
        grid=(NP // _MMBLK,),
        in_specs=[
            pl.BlockSpec((NC, _MMBLK, D), lambda i: (0, i, 0)),
            pl.BlockSpec((_MMBLK, D), lambda i: (i, 0)),
            pl.BlockSpec((NC * NP,), lambda i: (0,)),
        ],
        out_specs=pl.BlockSpec((_MMBLK, D), lambda i: (i, 0)),
        out_shape=jax.ShapeDtypeStruct((NP, D), jnp.float32),
    )(p, h, c)


def kernel(x, edge_index, W, b):
    dst = edge_index[0].reshape(NW, NB, 1, K)
    src = edge_index[1].reshape(NW, EPW)
    h0 = _matmul(x, W, b.reshape(1, D))
    p1, c = _sc_layer_cnt(h0, src, dst)
    h1 = _combine(p1, h0, c)
    (p2,) = _sc_layer(h1, src, dst)
    h2 = _combine(p2, h1, c)
    return h2[:N]
